# Initial kernel scaffold; baseline (speedup 1.0000x reference)
import jax
import jax.numpy as jnp
from jax.experimental import pallas as pl


def kernel(hidden_states, weight, bias):
    raise NotImplementedError("write your pallas kernel here")


# speedup vs baseline: 4.4325x; 4.4325x over previous
import jax
import jax.numpy as jnp
from jax.experimental import pallas as pl

_HIDDEN = 768
_EXPERTS = 8
_BLOCK = 2048


def _router_kernel(x_ref, w_ref, b_ref, scores_ref, idx_ref):
    x = x_ref[...]
    w = w_ref[...]
    b = b_ref[...]
    logits = jax.lax.dot_general(
        x, w, (((1,), (1,)), ((), ())),
        preferred_element_type=jnp.float32) + b

    e = jax.lax.broadcasted_iota(jnp.int32, logits.shape, 1)
    m1 = jnp.max(logits, axis=1, keepdims=True)
    a1 = jnp.min(jnp.where(logits == m1, e, _EXPERTS), axis=1, keepdims=True)
    masked = jnp.where(e == a1, -jnp.inf, logits)
    m2 = jnp.max(masked, axis=1, keepdims=True)
    a2 = jnp.min(jnp.where(masked == m2, e, _EXPERTS), axis=1, keepdims=True)

    e2 = jnp.exp(m2 - m1)
    denom = 1.0 + e2
    p1 = 1.0 / denom
    p2 = e2 / denom

    scores_ref[...] = jnp.where(e == a1, p1, jnp.where(e == a2, p2, 0.0))
    idx_ref[...] = jnp.concatenate([a1, a2], axis=1)


def kernel(hidden_states, weight, bias):
    x = hidden_states.reshape(-1, _HIDDEN)
    n = x.shape[0]
    grid = (n // _BLOCK,)
    scores, idx = pl.pallas_call(
        _router_kernel,
        grid=grid,
        in_specs=[
            pl.BlockSpec((_BLOCK, _HIDDEN), lambda i: (i, 0)),
            pl.BlockSpec((_EXPERTS, _HIDDEN), lambda i: (0, 0)),
            pl.BlockSpec((1, _EXPERTS), lambda i: (0, 0)),
        ],
        out_specs=[
            pl.BlockSpec((_BLOCK, _EXPERTS), lambda i: (i, 0)),
            pl.BlockSpec((_BLOCK, 2), lambda i: (i, 0)),
        ],
        out_shape=[
            jax.ShapeDtypeStruct((n, _EXPERTS), jnp.float32),
            jax.ShapeDtypeStruct((n, 2), jnp.int32),
        ],
    )(x, weight, bias.reshape(1, _EXPERTS))
    return scores, idx


# speedup vs baseline: 4.7659x; 1.0752x over previous
import jax
import jax.numpy as jnp
from jax.experimental import pallas as pl

_HIDDEN = 768
_EXPERTS = 8
_BLOCK = 4096


def _router_kernel(x_ref, w_ref, b_ref, scores_ref, idx_ref):
    x = x_ref[...]
    w = w_ref[...]
    b = b_ref[...]
    logits = jax.lax.dot_general(
        x, w, (((1,), (1,)), ((), ())),
        preferred_element_type=jnp.float32) + b

    e = jax.lax.broadcasted_iota(jnp.int32, logits.shape, 1)
    m1 = jnp.max(logits, axis=1, keepdims=True)
    a1 = jnp.min(jnp.where(logits == m1, e, _EXPERTS), axis=1, keepdims=True)
    masked = jnp.where(e == a1, -jnp.inf, logits)
    m2 = jnp.max(masked, axis=1, keepdims=True)
    a2 = jnp.min(jnp.where(masked == m2, e, _EXPERTS), axis=1, keepdims=True)

    e2 = jnp.exp(m2 - m1)
    denom = 1.0 + e2
    p1 = 1.0 / denom
    p2 = e2 / denom

    scores_ref[...] = jnp.where(e == a1, p1, jnp.where(e == a2, p2, 0.0))
    idx_ref[...] = jnp.concatenate([a1, a2], axis=1)


def kernel(hidden_states, weight, bias):
    x = hidden_states.reshape(-1, _HIDDEN)
    n = x.shape[0]
    grid = (n // _BLOCK,)
    scores, idx = pl.pallas_call(
        _router_kernel,
        grid=grid,
        in_specs=[
            pl.BlockSpec((_BLOCK, _HIDDEN), lambda i: (i, 0)),
            pl.BlockSpec((_EXPERTS, _HIDDEN), lambda i: (0, 0)),
            pl.BlockSpec((1, _EXPERTS), lambda i: (0, 0)),
        ],
        out_specs=[
            pl.BlockSpec((_BLOCK, _EXPERTS), lambda i: (i, 0)),
            pl.BlockSpec((_BLOCK, 2), lambda i: (i, 0)),
        ],
        out_shape=[
            jax.ShapeDtypeStruct((n, _EXPERTS), jnp.float32),
            jax.ShapeDtypeStruct((n, 2), jnp.int32),
        ],
    )(x, weight, bias.reshape(1, _EXPERTS))
    return scores, idx


# speedup vs baseline: 9.1141x; 1.9123x over previous
import jax
import jax.numpy as jnp
from jax.experimental import pallas as pl

_HIDDEN = 768
_EXPERTS = 8
_BLOCK = 4096


def _router_kernel(x_ref, w_ref, b_ref, scores_ref, idx_ref):
    x = x_ref[...]
    w = w_ref[...]
    b = b_ref[...]
    logits = jax.lax.dot_general(
        x, w, (((1,), (1,)), ((), ())),
        preferred_element_type=jnp.float32)
    lt = logits.T + b

    e = jax.lax.broadcasted_iota(jnp.int32, lt.shape, 0)
    m1 = jnp.max(lt, axis=0, keepdims=True)
    a1 = jnp.min(jnp.where(lt == m1, e, _EXPERTS), axis=0, keepdims=True)
    masked = jnp.where(e == a1, -jnp.inf, lt)
    m2 = jnp.max(masked, axis=0, keepdims=True)
    a2 = jnp.min(jnp.where(masked == m2, e, _EXPERTS), axis=0, keepdims=True)

    e2 = jnp.exp(m2 - m1)
    denom = 1.0 + e2
    p1 = 1.0 / denom
    p2 = e2 / denom

    scores_ref[...] = jnp.where(e == a1, p1, jnp.where(e == a2, p2, 0.0))
    idx_ref[...] = jnp.concatenate([a1, a2], axis=0)


def kernel(hidden_states, weight, bias):
    x = hidden_states.reshape(-1, _HIDDEN)
    n = x.shape[0]
    grid = (n // _BLOCK,)
    scores_t, idx_t = pl.pallas_call(
        _router_kernel,
        grid=grid,
        in_specs=[
            pl.BlockSpec((_BLOCK, _HIDDEN), lambda i: (i, 0)),
            pl.BlockSpec((_EXPERTS, _HIDDEN), lambda i: (0, 0)),
            pl.BlockSpec((_EXPERTS, 1), lambda i: (0, 0)),
        ],
        out_specs=[
            pl.BlockSpec((_EXPERTS, _BLOCK), lambda i: (0, i)),
            pl.BlockSpec((2, _BLOCK), lambda i: (0, i)),
        ],
        out_shape=[
            jax.ShapeDtypeStruct((_EXPERTS, n), jnp.float32),
            jax.ShapeDtypeStruct((2, n), jnp.int32),
        ],
    )(x, weight, bias.reshape(_EXPERTS, 1))
    return scores_t.T, idx_t.T


# speedup vs baseline: 9.4125x; 1.0327x over previous
import jax
import jax.numpy as jnp
from jax.experimental import pallas as pl

_HIDDEN = 768
_EXPERTS = 8
_BLOCK = 4096


def _router_kernel(x_ref, w_ref, b_ref, scores_ref, idx_ref):
    x = x_ref[...]
    w = w_ref[...]
    b = b_ref[...]
    logits = jax.lax.dot_general(
        x, w, (((1,), (1,)), ((), ())),
        preferred_element_type=jnp.float32)
    lt = logits.T + b.T

    e = jax.lax.broadcasted_iota(jnp.int32, lt.shape, 0)
    m1 = jnp.max(lt, axis=0, keepdims=True)
    a1 = jnp.min(jnp.where(lt == m1, e, _EXPERTS), axis=0, keepdims=True)
    masked = jnp.where(e == a1, -jnp.inf, lt)
    m2 = jnp.max(masked, axis=0, keepdims=True)
    a2 = jnp.min(jnp.where(masked == m2, e, _EXPERTS), axis=0, keepdims=True)

    e2 = jnp.exp(m2 - m1)
    denom = 1.0 + e2
    p1 = 1.0 / denom
    p2 = e2 / denom

    scores_ref[...] = jnp.where(e == a1, p1, jnp.where(e == a2, p2, 0.0))
    idx_ref[...] = jnp.concatenate([a1, a2], axis=0)


def kernel(hidden_states, weight, bias):
    x = hidden_states.reshape(-1, _HIDDEN)
    n = x.shape[0]
    grid = (n // _BLOCK,)
    scores_t, idx_t = pl.pallas_call(
        _router_kernel,
        grid=grid,
        in_specs=[
            pl.BlockSpec((_BLOCK, _HIDDEN), lambda i: (i, 0)),
            pl.BlockSpec((_EXPERTS, _HIDDEN), lambda i: (0, 0)),
            pl.BlockSpec((1, _EXPERTS), lambda i: (0, 0)),
        ],
        out_specs=[
            pl.BlockSpec((_EXPERTS, _BLOCK), lambda i: (0, i)),
            pl.BlockSpec((2, _BLOCK), lambda i: (0, i)),
        ],
        out_shape=[
            jax.ShapeDtypeStruct((_EXPERTS, n), jnp.float32),
            jax.ShapeDtypeStruct((2, n), jnp.int32),
        ],
    )(x, weight, bias.reshape(1, _EXPERTS))
    return scores_t.T, idx_t.T
